# baseline (speedup 1.0000x reference)
import functools

import jax
import jax.numpy as jnp
from jax import lax
from jax.experimental import pallas as pl
from jax.experimental.pallas import tpu as pltpu
from jax.experimental.pallas import tpu_sc as plsc

N = 16384
CHUNK = 128
_info = plsc.get_sparse_core_info()
_NC = _info.num_cores
_NS = _info.num_subcores
_NW = _NC * _NS
_PER_W = N // _NW
_N_STREAMS = _PER_W // CHUNK


def _sc_gather_body(table_hbm, idx_hbm, out_hbm, idx_v, vals_v, sem):
    wid = lax.axis_index("s") * _NC + lax.axis_index("c")
    base = wid * _PER_W
    pltpu.sync_copy(idx_hbm.at[pl.ds(base, _PER_W)], idx_v)
    pltpu.async_copy(table_hbm.at[idx_v], vals_v, sem).wait()
    pltpu.sync_copy(vals_v, out_hbm.at[pl.ds(base, _PER_W)])


_sc_gather = functools.partial(
    pl.kernel,
    mesh=plsc.VectorSubcoreMesh(core_axis_name="c", subcore_axis_name="s"),
    out_type=jax.ShapeDtypeStruct((N,), jnp.float32),
    scratch_types=[
        pltpu.VMEM((_PER_W,), jnp.int32),
        pltpu.VMEM((_PER_W,), jnp.float32),
        pltpu.SemaphoreType.DMA,
    ],
)(_sc_gather_body)


def _tc_logp_body(p_ref, o_ref):
    o_ref[...] = jnp.log(p_ref[...])


def _tc_logp(p0):
    return pl.pallas_call(
        _tc_logp_body,
        out_shape=jax.ShapeDtypeStruct((N,), jnp.float32),
    )(p0)


def _tc_math_body(t_ref, p_ref, lp_ref, g_ref, o_ref):
    g = g_ref[...]
    e = jnp.exp(g)
    pe = p_ref[...] * e
    s1 = jnp.sum(pe)
    s2 = jnp.sum(pe * e)
    two = t_ref[0] >= 22
    k = jnp.where(two, 2.0, 1.0)
    s = jnp.where(two, s2, s1)
    o_ref[...] = lp_ref[...] + k * g - jnp.log(s)


def _tc_math(t, p0, lp, g):
    return pl.pallas_call(
        _tc_math_body,
        out_shape=jax.ShapeDtypeStruct((N,), jnp.float32),
        in_specs=[
            pl.BlockSpec(memory_space=pltpu.SMEM),
            pl.BlockSpec(memory_space=pltpu.VMEM),
            pl.BlockSpec(memory_space=pltpu.VMEM),
            pl.BlockSpec(memory_space=pltpu.VMEM),
        ],
    )(t, p0, lp, g)


def kernel(p0, nonzero_idxs, t_idx, fitness):
    idx = nonzero_idxs.astype(jnp.int32)
    g = _sc_gather(fitness, idx)
    lp = _tc_logp(p0)
    t = jnp.asarray(t_idx, jnp.int32).reshape(1)
    return _tc_math(t, p0, lp, g)


# speedup vs baseline: 1.0061x; 1.0061x over previous
import functools

import jax
import jax.numpy as jnp
from jax import lax
from jax.experimental import pallas as pl
from jax.experimental.pallas import tpu as pltpu
from jax.experimental.pallas import tpu_sc as plsc

N = 16384
CHUNK = 128
_info = plsc.get_sparse_core_info()
_NC = _info.num_cores
_NS = _info.num_subcores
_NW = _NC * _NS
_PER_W = N // _NW
_N_STREAMS = _PER_W // CHUNK


def _sc_gather_body(table_hbm, idx_hbm, out_hbm, idx_v, vals_v, sem_i, sem_g, sem_o):
    wid = lax.axis_index("s") * _NC + lax.axis_index("c")
    base = wid * _PER_W
    icps = []
    for j in range(_N_STREAMS):
        icps.append(pltpu.async_copy(
            idx_hbm.at[pl.ds(base + j * CHUNK, CHUNK)],
            idx_v.at[pl.ds(j * CHUNK, CHUNK)], sem_i))
    gcps = []
    for j in range(_N_STREAMS):
        icps[j].wait()
        gcps.append(pltpu.async_copy(
            table_hbm.at[idx_v.at[pl.ds(j * CHUNK, CHUNK)]],
            vals_v.at[pl.ds(j * CHUNK, CHUNK)], sem_g))
    ocps = []
    for j in range(_N_STREAMS):
        gcps[j].wait()
        ocps.append(pltpu.async_copy(
            vals_v.at[pl.ds(j * CHUNK, CHUNK)],
            out_hbm.at[pl.ds(base + j * CHUNK, CHUNK)], sem_o))
    for cp in ocps:
        cp.wait()


_sc_gather = functools.partial(
    pl.kernel,
    mesh=plsc.VectorSubcoreMesh(core_axis_name="c", subcore_axis_name="s"),
    out_type=jax.ShapeDtypeStruct((N,), jnp.float32),
    scratch_types=[
        pltpu.VMEM((_PER_W,), jnp.int32),
        pltpu.VMEM((_PER_W,), jnp.float32),
        pltpu.SemaphoreType.DMA,
        pltpu.SemaphoreType.DMA,
        pltpu.SemaphoreType.DMA,
    ],
)(_sc_gather_body)


def _tc_logp_body(p_ref, o_ref):
    o_ref[...] = jnp.log(p_ref[...])


def _tc_logp(p0):
    return pl.pallas_call(
        _tc_logp_body,
        out_shape=jax.ShapeDtypeStruct((N,), jnp.float32),
    )(p0)


def _tc_math_body(t_ref, p_ref, lp_ref, g_ref, o_ref):
    g = g_ref[...]
    e = jnp.exp(g)
    pe = p_ref[...] * e
    s1 = jnp.sum(pe)
    s2 = jnp.sum(pe * e)
    two = t_ref[0] >= 22
    k = jnp.where(two, 2.0, 1.0)
    s = jnp.where(two, s2, s1)
    o_ref[...] = lp_ref[...] + k * g - jnp.log(s)


def _tc_math(t, p0, lp, g):
    return pl.pallas_call(
        _tc_math_body,
        out_shape=jax.ShapeDtypeStruct((N,), jnp.float32),
        in_specs=[
            pl.BlockSpec(memory_space=pltpu.SMEM),
            pl.BlockSpec(memory_space=pltpu.VMEM),
            pl.BlockSpec(memory_space=pltpu.VMEM),
            pl.BlockSpec(memory_space=pltpu.VMEM),
        ],
    )(t, p0, lp, g)


def kernel(p0, nonzero_idxs, t_idx, fitness):
    idx = nonzero_idxs.astype(jnp.int32)
    g = _sc_gather(fitness, idx)
    lp = _tc_logp(p0)
    t = jnp.asarray(t_idx, jnp.int32).reshape(1)
    return _tc_math(t, p0, lp, g)
